# Initial kernel scaffold; baseline (speedup 1.0000x reference)
import jax
import jax.numpy as jnp
from jax.experimental import pallas as pl


def kernel(features, image_width, image_height, W1, b1, Wobj, bobj, Wt, bt):
    raise NotImplementedError("write your pallas kernel here")


# speedup vs baseline: 1.0014x; 1.0014x over previous
import functools

import jax
import jax.numpy as jnp
import numpy as np
from jax.experimental import pallas as pl
from jax.experimental.pallas import tpu as pltpu

_P = 2500
_NA = 9
_H = 112
_CO = 512
_CI = 1024
_COT = 128
_OFFS = [(dy, dx) for dy in (-1, 0, 1) for dx in (-1, 0, 1)]


def _anchor_consts(num_x, num_y):
    ys = np.linspace(0.0, 800.0, num_y + 2)[1:-1]
    xs = np.linspace(0.0, 800.0, num_x + 2)[1:-1]
    whs = []
    for (r0, r1) in [(1, 2), (1, 1), (2, 1)]:
        for size in [32, 64, 128]:
            r = r0 / r1
            whs.append((size * np.sqrt(1.0 / r), size * np.sqrt(r)))
    whs = np.asarray(whs)
    P = num_x * num_y
    cx = np.tile(xs, num_y)
    cy = np.repeat(ys, num_x)
    cb = np.zeros((P * _NA, 4), np.float32)
    cb[:, 0] = np.repeat(cx, _NA)
    cb[:, 1] = np.repeat(cy, _NA)
    cb[:, 2] = np.tile(whs[:, 0], P)
    cb[:, 3] = np.tile(whs[:, 1], P)
    x1 = cb[:, 0] - cb[:, 2] / np.float32(2.0)
    y1 = cb[:, 1] - cb[:, 3] / np.float32(2.0)
    x2 = cb[:, 0] + cb[:, 2] / np.float32(2.0)
    y2 = cb[:, 1] + cb[:, 3] / np.float32(2.0)
    anchors = np.stack([x1, y1, x2, y2], axis=1)
    x1g = x1.reshape(P, _NA).T
    y1g = y1.reshape(P, _NA).T
    x2g = x2.reshape(P, _NA).T
    y2g = y2.reshape(P, _NA).T
    awg = x2g - x1g
    ahg = y2g - y1g
    acx = x1g + awg / np.float32(2.0)
    acy = y1g + ahg / np.float32(2.0)
    anc = np.zeros((64, P), np.float32)
    anc[0:9] = acx
    anc[16:25] = acy
    anc[32:41] = awg
    anc[48:57] = ahg
    return jnp.asarray(anchors), jnp.asarray(anc)


def _rpn_body(x2, wm, wh, b1r, bhr, anc, iwih, heads, boxes, key, f_s):
    i = pl.program_id(0)
    f_s[...] = jnp.broadcast_to(b1r[...], (_COT, _P))
    for k, (dy, dx) in enumerate(_OFFS):
        z = jax.lax.dot_general(wm[k], x2[...], (((1,), (0,)), ((), ())),
                                preferred_element_type=jnp.float32)
        off = 50 * dy + dx
        dst0 = max(0, -50 * dy, -off)
        dst1 = min(2500, 2500 - 50 * dy, 2500 - off)
        L = dst1 - dst0
        src0 = dst0 + off
        if dx == 0:
            f_s[:, dst0:dst1] += z[:, src0:src0 + L]
        else:
            col = jax.lax.broadcasted_iota(jnp.int32, (1, L), 1) + dst0
            ix = jax.lax.rem(col, 50)
            ok = jnp.logical_and(ix + dx >= 0, ix + dx <= 49)
            f_s[:, dst0:dst1] += jnp.where(ok, z[:, src0:src0 + L], 0.0)
    f = jnp.maximum(f_s[...], 0.0)
    contrib = jax.lax.dot_general(wh[...], f, (((1,), (0,)), ((), ())),
                                  preferred_element_type=jnp.float32)

    @pl.when(i == 0)
    def _():
        heads[...] = jnp.broadcast_to(bhr[...], (_H, _P)) + contrib

    @pl.when(i > 0)
    def _():
        heads[...] += contrib

    @pl.when(i == pl.num_programs(0) - 1)
    def _():
        h = heads[...]
        tx = h[32:41]
        ty = h[48:57]
        tw = h[64:73]
        th = h[80:89]
        sc = h[96:105]
        acx = anc[0:9]
        acy = anc[16:25]
        aw = anc[32:41]
        ah = anc[48:57]
        ncx = acx + tx * aw
        ncy = acy + ty * ah
        nw = aw * jnp.exp(tw)
        nh = ah * jnp.exp(th)
        iw = iwih[0, 0]
        ih = iwih[0, 1]
        x1 = jnp.clip(ncx - nw * 0.5, 0.0, iw)
        y1 = jnp.clip(ncy - nh * 0.5, 0.0, ih)
        x2c = jnp.clip(ncx + nw * 0.5, 0.0, iw)
        y2c = jnp.clip(ncy + nh * 0.5, 0.0, ih)
        valid = jnp.logical_and(x2c - x1 >= 16.0, y2c - y1 >= 16.0)
        smax = jnp.max(sc)
        smin = jnp.min(sc)
        keyf = jnp.where(valid, sc, sc - (smax - smin + 1.0))
        boxes[0:9] = x1
        boxes[16:25] = y1
        boxes[32:41] = x2c
        boxes[48:57] = y2c
        key[0:9] = keyf


def _rpn_call(x2, wm, wh, b1r, bhr, anc, iwih):
    return pl.pallas_call(
        _rpn_body,
        grid=(4,),
        in_specs=[
            pl.BlockSpec((_CI, _P), lambda i: (0, 0)),
            pl.BlockSpec((9, _COT, _CI), lambda i: (0, i, 0)),
            pl.BlockSpec((_H, _COT), lambda i: (0, i)),
            pl.BlockSpec((_COT, 1), lambda i: (i, 0)),
            pl.BlockSpec((_H, 1), lambda i: (0, 0)),
            pl.BlockSpec((64, _P), lambda i: (0, 0)),
            pl.BlockSpec((1, 2), lambda i: (0, 0)),
        ],
        out_specs=[
            pl.BlockSpec((_H, _P), lambda i: (0, 0)),
            pl.BlockSpec((64, _P), lambda i: (0, 0)),
            pl.BlockSpec((16, _P), lambda i: (0, 0)),
        ],
        out_shape=[
            jax.ShapeDtypeStruct((_H, _P), jnp.float32),
            jax.ShapeDtypeStruct((64, _P), jnp.float32),
            jax.ShapeDtypeStruct((16, _P), jnp.float32),
        ],
        scratch_shapes=[pltpu.VMEM((_COT, _P), jnp.float32)],
        compiler_params=pltpu.CompilerParams(
            dimension_semantics=("arbitrary",)),
    )(x2, wm, wh, b1r, bhr, anc, iwih)


def _nms_keep(boxes, thresh):
    n = boxes.shape[0]
    x1, y1, x2, y2 = boxes[:, 0], boxes[:, 1], boxes[:, 2], boxes[:, 3]
    areas = (x2 - x1) * (y2 - y1)
    idxs = jnp.arange(n)

    def body(i, keep):
        xi1 = jnp.maximum(x1[i], x1)
        yi1 = jnp.maximum(y1[i], y1)
        xi2 = jnp.minimum(x2[i], x2)
        yi2 = jnp.minimum(y2[i], y2)
        inter = jnp.maximum(xi2 - xi1, 0.0) * jnp.maximum(yi2 - yi1, 0.0)
        iou = inter / (areas[i] + areas - inter + 1e-8)
        sup = (iou > thresh) & (idxs > i) & keep[i]
        return keep & (~sup)

    return jax.lax.fori_loop(0, n, body, jnp.ones((n,), dtype=bool))


def kernel(features, image_width, image_height, W1, b1, Wobj, bobj, Wt, bt):
    num_y, num_x = features.shape[2], features.shape[3]
    anchors, anc = _anchor_consts(num_x, num_y)

    x2 = features.reshape(_CI, _P)
    wm = W1.transpose(2, 3, 0, 1).reshape(9, _CO, _CI)
    wo = Wobj.reshape(18, _CO)
    wt = Wt.reshape(36, _CO)
    wh = jnp.zeros((_H, _CO), jnp.float32)
    wh = wh.at[0:18].set(wo)
    wh = wh.at[32:41].set(wt[0::4])
    wh = wh.at[48:57].set(wt[1::4])
    wh = wh.at[64:73].set(wt[2::4])
    wh = wh.at[80:89].set(wt[3::4])
    wh = wh.at[96:105].set(wo[1::2])
    bh = jnp.zeros((_H,), jnp.float32)
    bh = bh.at[0:18].set(bobj)
    bh = bh.at[32:41].set(bt[0::4])
    bh = bh.at[48:57].set(bt[1::4])
    bh = bh.at[64:73].set(bt[2::4])
    bh = bh.at[80:89].set(bt[3::4])
    bh = bh.at[96:105].set(bobj[1::2])
    b1r = b1.reshape(_CO, 1)
    bhr = bh.reshape(_H, 1)
    iwih = jnp.stack([jnp.asarray(image_width, jnp.float32),
                      jnp.asarray(image_height, jnp.float32)]).reshape(1, 2)

    heads, boxesg, keyg = _rpn_call(x2, wm, wh, b1r, bhr, anc, iwih)

    obj2 = heads[0:18].T.reshape(_P * _NA, 2)
    trn2 = jnp.stack([heads[32:41], heads[48:57], heads[64:73],
                      heads[80:89]]).transpose(2, 1, 0).reshape(_P * _NA, 4)
    boxesf = jnp.stack([boxesg[0:9], boxesg[16:25], boxesg[32:41],
                        boxesg[48:57]]).transpose(2, 1, 0).reshape(_P * _NA, 4)
    keyf = keyg[0:9].T.reshape(_P * _NA)

    order = jnp.argsort(-keyf, stable=True)
    b6 = boxesf[order[:6000]]
    keep = _nms_keep(b6, 0.7)
    ord3 = jnp.argsort(jnp.where(keep, 0, 1), stable=True)
    props = b6[ord3][:300]
    return (anchors, obj2, trn2, props)


# speedup vs baseline: 218.4716x; 218.1687x over previous
import functools

import jax
import jax.numpy as jnp
import numpy as np
from jax.experimental import pallas as pl
from jax.experimental.pallas import tpu as pltpu

_P = 2500
_NA = 9
_H = 112
_CO = 512
_CI = 1024
_COT = 128
_OFFS = [(dy, dx) for dy in (-1, 0, 1) for dx in (-1, 0, 1)]


def _anchor_consts(num_x, num_y):
    ys = np.linspace(0.0, 800.0, num_y + 2)[1:-1]
    xs = np.linspace(0.0, 800.0, num_x + 2)[1:-1]
    whs = []
    for (r0, r1) in [(1, 2), (1, 1), (2, 1)]:
        for size in [32, 64, 128]:
            r = r0 / r1
            whs.append((size * np.sqrt(1.0 / r), size * np.sqrt(r)))
    whs = np.asarray(whs)
    P = num_x * num_y
    cx = np.tile(xs, num_y)
    cy = np.repeat(ys, num_x)
    cb = np.zeros((P * _NA, 4), np.float32)
    cb[:, 0] = np.repeat(cx, _NA)
    cb[:, 1] = np.repeat(cy, _NA)
    cb[:, 2] = np.tile(whs[:, 0], P)
    cb[:, 3] = np.tile(whs[:, 1], P)
    x1 = cb[:, 0] - cb[:, 2] / np.float32(2.0)
    y1 = cb[:, 1] - cb[:, 3] / np.float32(2.0)
    x2 = cb[:, 0] + cb[:, 2] / np.float32(2.0)
    y2 = cb[:, 1] + cb[:, 3] / np.float32(2.0)
    anchors = np.stack([x1, y1, x2, y2], axis=1)
    x1g = x1.reshape(P, _NA).T
    y1g = y1.reshape(P, _NA).T
    x2g = x2.reshape(P, _NA).T
    y2g = y2.reshape(P, _NA).T
    awg = x2g - x1g
    ahg = y2g - y1g
    acx = x1g + awg / np.float32(2.0)
    acy = y1g + ahg / np.float32(2.0)
    anc = np.zeros((64, P), np.float32)
    anc[0:9] = acx
    anc[16:25] = acy
    anc[32:41] = awg
    anc[48:57] = ahg
    return jnp.asarray(anchors), jnp.asarray(anc)


def _rpn_body(x2, wm, wh, b1r, bhr, anc, iwih, heads, boxes, key, f_s):
    i = pl.program_id(0)
    f_s[...] = jnp.broadcast_to(b1r[...], (_COT, _P))
    for k, (dy, dx) in enumerate(_OFFS):
        z = jax.lax.dot_general(wm[k], x2[...], (((1,), (0,)), ((), ())),
                                preferred_element_type=jnp.float32)
        off = 50 * dy + dx
        dst0 = max(0, -50 * dy, -off)
        dst1 = min(2500, 2500 - 50 * dy, 2500 - off)
        L = dst1 - dst0
        src0 = dst0 + off
        if dx == 0:
            f_s[:, dst0:dst1] += z[:, src0:src0 + L]
        else:
            col = jax.lax.broadcasted_iota(jnp.int32, (1, L), 1) + dst0
            ix = jax.lax.rem(col, 50)
            ok = jnp.logical_and(ix + dx >= 0, ix + dx <= 49)
            f_s[:, dst0:dst1] += jnp.where(ok, z[:, src0:src0 + L], 0.0)
    f = jnp.maximum(f_s[...], 0.0)
    contrib = jax.lax.dot_general(wh[...], f, (((1,), (0,)), ((), ())),
                                  preferred_element_type=jnp.float32)

    @pl.when(i == 0)
    def _():
        heads[...] = jnp.broadcast_to(bhr[...], (_H, _P)) + contrib

    @pl.when(i > 0)
    def _():
        heads[...] += contrib

    @pl.when(i == pl.num_programs(0) - 1)
    def _():
        h = heads[...]
        tx = h[32:41]
        ty = h[48:57]
        tw = h[64:73]
        th = h[80:89]
        sc = h[96:105]
        acx = anc[0:9]
        acy = anc[16:25]
        aw = anc[32:41]
        ah = anc[48:57]
        ncx = acx + tx * aw
        ncy = acy + ty * ah
        nw = aw * jnp.exp(tw)
        nh = ah * jnp.exp(th)
        iw = iwih[0, 0]
        ih = iwih[0, 1]
        x1 = jnp.clip(ncx - nw * 0.5, 0.0, iw)
        y1 = jnp.clip(ncy - nh * 0.5, 0.0, ih)
        x2c = jnp.clip(ncx + nw * 0.5, 0.0, iw)
        y2c = jnp.clip(ncy + nh * 0.5, 0.0, ih)
        valid = jnp.logical_and(x2c - x1 >= 16.0, y2c - y1 >= 16.0)
        smax = jnp.max(sc)
        smin = jnp.min(sc)
        keyf = jnp.where(valid, sc, sc - (smax - smin + 1.0))
        boxes[0:9] = x1
        boxes[16:25] = y1
        boxes[32:41] = x2c
        boxes[48:57] = y2c
        key[0:9] = keyf


def _rpn_call(x2, wm, wh, b1r, bhr, anc, iwih):
    return pl.pallas_call(
        _rpn_body,
        grid=(4,),
        in_specs=[
            pl.BlockSpec((_CI, _P), lambda i: (0, 0)),
            pl.BlockSpec((9, _COT, _CI), lambda i: (0, i, 0)),
            pl.BlockSpec((_H, _COT), lambda i: (0, i)),
            pl.BlockSpec((_COT, 1), lambda i: (i, 0)),
            pl.BlockSpec((_H, 1), lambda i: (0, 0)),
            pl.BlockSpec((64, _P), lambda i: (0, 0)),
            pl.BlockSpec((1, 2), lambda i: (0, 0)),
        ],
        out_specs=[
            pl.BlockSpec((_H, _P), lambda i: (0, 0)),
            pl.BlockSpec((64, _P), lambda i: (0, 0)),
            pl.BlockSpec((16, _P), lambda i: (0, 0)),
        ],
        out_shape=[
            jax.ShapeDtypeStruct((_H, _P), jnp.float32),
            jax.ShapeDtypeStruct((64, _P), jnp.float32),
            jax.ShapeDtypeStruct((16, _P), jnp.float32),
        ],
        scratch_shapes=[pltpu.VMEM((_COT, _P), jnp.float32)],
        compiler_params=pltpu.CompilerParams(
            dimension_semantics=("arbitrary",)),
    )(x2, wm, wh, b1r, bhr, anc, iwih)


_NN = 6016
_NB = 47
_BK = 128


def _nms_body(x1r, y1r, x2r, y2r, x1t, y1t, x2t, y2t, keep):
    keep[...] = jnp.ones((1, _NN), jnp.float32)
    lane = jax.lax.broadcasted_iota(jnp.int32, (1, _NN), 1)
    xr1 = x1r[...]
    yr1 = y1r[...]
    xr2 = x2r[...]
    yr2 = y2r[...]
    arow = (xr2 - xr1) * (yr2 - yr1)
    rit = jax.lax.broadcasted_iota(jnp.int32, (_BK, _BK), 0)
    cit = jax.lax.broadcasted_iota(jnp.int32, (_BK, _BK), 1)
    tri = (rit < cit).astype(jnp.float32)

    xt1 = x1t[...]
    yt1 = y1t[...]
    xt2 = x2t[...]
    yt2 = y2t[...]

    def blk_body(state):
        b, cnt = state
        oh = (jax.lax.broadcasted_iota(jnp.int32, (1, _NB), 1) == b
              ).astype(jnp.float32)
        xb1 = jnp.sum(xt1 * oh, axis=1, keepdims=True)
        yb1 = jnp.sum(yt1 * oh, axis=1, keepdims=True)
        xb2 = jnp.sum(xt2 * oh, axis=1, keepdims=True)
        yb2 = jnp.sum(yt2 * oh, axis=1, keepdims=True)
        ab = (xb2 - xb1) * (yb2 - yb1)
        bs = pl.multiple_of(b * _BK, _BK)
        inter = (jnp.maximum(jnp.minimum(xb2, xr2) - jnp.maximum(xb1, xr1), 0.0)
                 * jnp.maximum(jnp.minimum(yb2, yr2) - jnp.maximum(yb1, yr1), 0.0))
        iou = inter / (ab + arow - inter + 1e-8)
        mf = jnp.where(iou > 0.7, 1.0, 0.0)
        xs1 = x1r[:, pl.ds(bs, _BK)]
        ys1 = y1r[:, pl.ds(bs, _BK)]
        xs2 = x2r[:, pl.ds(bs, _BK)]
        ys2 = y2r[:, pl.ds(bs, _BK)]
        asr = (xs2 - xs1) * (ys2 - ys1)
        ibb = (jnp.maximum(jnp.minimum(xb2, xs2) - jnp.maximum(xb1, xs1), 0.0)
               * jnp.maximum(jnp.minimum(yb2, ys2) - jnp.maximum(yb1, ys1), 0.0))
        ioubb = ibb / (ab + asr - ibb + 1e-8)
        mbb = jnp.where(ioubb > 0.7, 1.0, 0.0) * tri
        ext = keep[:, pl.ds(bs, _BK)]

        def fix_cond(fs):
            _, ch, it = fs
            return jnp.logical_and(ch, it < _BK + 2)

        def fix_body(fs):
            k, _, it = fs
            s = jax.lax.dot_general(k, mbb, (((1,), (0,)), ((), ())),
                                    preferred_element_type=jnp.float32)
            kn = ext * jnp.where(s >= 0.5, 0.0, 1.0)
            ch = jnp.sum(jnp.abs(kn - k)) > 0.0
            return kn, ch, it + 1

        kb, _, _ = jax.lax.while_loop(fix_cond, fix_body,
                                      (ext, jnp.bool_(True), jnp.int32(0)))
        keep[:, pl.ds(bs, _BK)] = kb
        s_all = jax.lax.dot_general(kb, mf, (((1,), (0,)), ((), ())),
                                    preferred_element_type=jnp.float32)
        sup = jnp.logical_and(s_all >= 0.5, lane >= (b + 1) * _BK)
        keep[...] = keep[...] * jnp.where(sup, 0.0, 1.0)
        gl = jax.lax.broadcasted_iota(jnp.int32, (1, _BK), 1) + b * _BK
        cnt = cnt + jnp.sum(kb * (gl < 6000).astype(jnp.float32))
        return b + 1, cnt

    def blk_cond(state):
        b, cnt = state
        return jnp.logical_and(b < _NB, cnt < 300.0)

    jax.lax.while_loop(blk_cond, blk_body, (jnp.int32(0), jnp.float32(0.0)))


def _nms_call(b6p):
    x1 = b6p[:, 0]
    y1 = b6p[:, 1]
    x2 = b6p[:, 2]
    y2 = b6p[:, 3]
    args = [x1.reshape(1, _NN), y1.reshape(1, _NN),
            x2.reshape(1, _NN), y2.reshape(1, _NN),
            x1.reshape(_NB, _BK).T, y1.reshape(_NB, _BK).T,
            x2.reshape(_NB, _BK).T, y2.reshape(_NB, _BK).T]
    return pl.pallas_call(
        _nms_body,
        out_shape=jax.ShapeDtypeStruct((1, _NN), jnp.float32),
    )(*args)


def kernel(features, image_width, image_height, W1, b1, Wobj, bobj, Wt, bt):
    num_y, num_x = features.shape[2], features.shape[3]
    anchors, anc = _anchor_consts(num_x, num_y)

    x2 = features.reshape(_CI, _P)
    wm = W1.transpose(2, 3, 0, 1).reshape(9, _CO, _CI)
    wo = Wobj.reshape(18, _CO)
    wt = Wt.reshape(36, _CO)
    wh = jnp.zeros((_H, _CO), jnp.float32)
    wh = wh.at[0:18].set(wo)
    wh = wh.at[32:41].set(wt[0::4])
    wh = wh.at[48:57].set(wt[1::4])
    wh = wh.at[64:73].set(wt[2::4])
    wh = wh.at[80:89].set(wt[3::4])
    wh = wh.at[96:105].set(wo[1::2])
    bh = jnp.zeros((_H,), jnp.float32)
    bh = bh.at[0:18].set(bobj)
    bh = bh.at[32:41].set(bt[0::4])
    bh = bh.at[48:57].set(bt[1::4])
    bh = bh.at[64:73].set(bt[2::4])
    bh = bh.at[80:89].set(bt[3::4])
    bh = bh.at[96:105].set(bobj[1::2])
    b1r = b1.reshape(_CO, 1)
    bhr = bh.reshape(_H, 1)
    iwih = jnp.stack([jnp.asarray(image_width, jnp.float32),
                      jnp.asarray(image_height, jnp.float32)]).reshape(1, 2)

    heads, boxesg, keyg = _rpn_call(x2, wm, wh, b1r, bhr, anc, iwih)

    obj2 = heads[0:18].T.reshape(_P * _NA, 2)
    trn2 = jnp.stack([heads[32:41], heads[48:57], heads[64:73],
                      heads[80:89]]).transpose(2, 1, 0).reshape(_P * _NA, 4)
    boxesf = jnp.stack([boxesg[0:9], boxesg[16:25], boxesg[32:41],
                        boxesg[48:57]]).transpose(2, 1, 0).reshape(_P * _NA, 4)
    keyf = keyg[0:9].T.reshape(_P * _NA)

    order = jnp.argsort(-keyf, stable=True)
    b6 = boxesf[order[:6000]]
    b6p = jnp.concatenate([b6, jnp.zeros((_NN - 6000, 4), jnp.float32)], axis=0)
    keepf = _nms_call(b6p)
    keep = keepf[0, :6000] > 0.5
    ord3 = jnp.argsort(jnp.where(keep, 0, 1), stable=True)
    props = b6[ord3][:300]
    return (anchors, obj2, trn2, props)
